# Initial kernel scaffold; baseline (speedup 1.0000x reference)
import functools

import jax
import jax.numpy as jnp
from jax import lax
from jax.experimental import pallas as pl
from jax.experimental.pallas import tpu as pltpu
from jax.experimental.pallas import tpu_sc as plsc

NNODE = 10000
NP = 10496
TRASH = NP - 1
NEDGE = 320000
EP = 327680
ECH = EP // 128
HD = 64
NCLS = 40
KSTEPS = 4
ALPHA = 0.1

_mesh = plsc.VectorSubcoreMesh(core_axis_name="c", subcore_axis_name="s")


def _rsqrt16(d):
    one = jnp.full((16,), 1, jnp.int32)
    yi = jnp.full((16,), 0x5F3759DF, jnp.int32) - lax.shift_right_logical(
        plsc.bitcast(d, jnp.int32), one)
    y = plsc.bitcast(yi, jnp.float32)
    for _ in range(3):
        y = y * (1.5 - 0.5 * d * y * y)
    return y


def _linin_body(x_ref, w_ref, b_ref, o_ref):
    o_ref[...] = jnp.dot(x_ref[...], w_ref[...],
                         preferred_element_type=jnp.float32) + b_ref[...]


_linin = pl.pallas_call(
    _linin_body,
    out_shape=jax.ShapeDtypeStruct((NP, HD), jnp.float32),
)


def _upd_body(p_ref, g_ref, a_ref, g0_ref, o_ref):
    s = p_ref[0] + p_ref[1] + g_ref[...]
    o_ref[...] = a_ref[:, 0:1] * s + ALPHA * g0_ref[...]


_upd = pl.pallas_call(
    _upd_body,
    out_shape=jax.ShapeDtypeStruct((NP, HD), jnp.float32),
)


def _fin_body(p_ref, g_ref, a_ref, g0_ref, inv_ref, w_ref, b_ref, o_ref):
    s = p_ref[0] + p_ref[1] + g_ref[...]
    h = (a_ref[:, 0:1] * s + ALPHA * g0_ref[...]) * inv_ref[:, 0:1]
    o_ref[...] = jnp.dot(jnp.maximum(h, 0.0), w_ref[...],
                         preferred_element_type=jnp.float32) + b_ref[...]


_fin = pl.pallas_call(
    _fin_body,
    out_shape=jax.ShapeDtypeStruct((NP, NCLS), jnp.float32),
)


@functools.partial(
    pl.kernel,
    out_type=[
        jax.ShapeDtypeStruct((NP, HD), jnp.float32),
        jax.ShapeDtypeStruct((NP, 16), jnp.float32),
        jax.ShapeDtypeStruct((NP, 16), jnp.float32),
    ],
    mesh=_mesh,
    scratch_types=[
        pltpu.VMEM_SHARED((NP, 16), jnp.float32),
        pltpu.VMEM((160, 128), jnp.int32),
        pltpu.VMEM((128, 16), jnp.float32),
        pltpu.VMEM((328, 16), jnp.float32),
        pltpu.VMEM((328, HD), jnp.float32),
        pltpu.VMEM((328, 16), jnp.float32),
        pltpu.VMEM((328, 16), jnp.float32),
    ],
)
def _prep_sc(col_hbm, h0_hbm, g0_hbm, a_hbm, inv_hbm,
             deg_sh, col_v, ones_v, deg_v, h_v, a_v, inv_v):
    s = lax.axis_index("s")
    c = lax.axis_index("c")
    wid = s * 2 + c

    one = jnp.full((16,), 1.0, jnp.float32)
    zero = jnp.zeros((16,), jnp.float32)

    @pl.loop(0, 128)
    def _(r):
        ones_v[r] = one

    @pl.loop(0, 328)
    def _(r):
        deg_v[r] = zero

    pltpu.sync_copy(deg_v, deg_sh.at[pl.ds(s * 656, 328)])
    pltpu.sync_copy(deg_v, deg_sh.at[pl.ds(s * 656 + 328, 328)])
    plsc.subcore_barrier()

    pltpu.sync_copy(col_hbm.at[pl.ds(s * 160, 160)], col_v)

    @pl.loop(0, 160)
    def _(j):
        pltpu.sync_copy(ones_v, deg_sh.at[col_v.at[j]], add=True)

    plsc.subcore_barrier()

    nb = wid * 328
    pltpu.sync_copy(deg_sh.at[pl.ds(nb, 328)], deg_v)
    pltpu.sync_copy(h0_hbm.at[pl.ds(nb, 328)], h_v)

    @pl.loop(0, 328)
    def _(r):
        d = deg_v[r] + 1.0
        y = _rsqrt16(d)
        a_v[r] = 0.9 * (y * y)
        inv_v[r] = d * y
        for f in range(4):
            h_v[r, pl.ds(f * 16, 16)] = h_v[r, pl.ds(f * 16, 16)] * y

    pltpu.sync_copy(a_v, a_hbm.at[pl.ds(nb, 328)])
    pltpu.sync_copy(inv_v, inv_hbm.at[pl.ds(nb, 328)])
    pltpu.sync_copy(h_v, g0_hbm.at[pl.ds(nb, 328)])


@functools.partial(
    pl.kernel,
    out_type=jax.ShapeDtypeStruct((2, NP, HD), jnp.float32),
    mesh=_mesh,
    scratch_types=[
        pltpu.VMEM_SHARED((NP, HD), jnp.float32),
        pltpu.VMEM((80, 128), jnp.int32),
        pltpu.VMEM((80, 128), jnp.int32),
        pltpu.VMEM((128, HD), jnp.float32),
        pltpu.SemaphoreType.DMA,
    ],
)
def _step_sc(g_hbm, row_hbm, col_hbm, part_hbm,
             part_sh, row_v, col_v, buf, sem):
    s = lax.axis_index("s")
    c = lax.axis_index("c")
    wid = s * 2 + c

    zero = jnp.zeros((16,), jnp.float32)

    @pl.loop(0, 128)
    def _(r):
        for f in range(4):
            buf[r, pl.ds(f * 16, 16)] = zero

    nb = s * 656

    @pl.loop(0, 5)
    def _(k):
        pltpu.sync_copy(buf, part_sh.at[pl.ds(nb + k * 128, 128)])

    pltpu.sync_copy(buf.at[pl.ds(0, 16)], part_sh.at[pl.ds(nb + 640, 16)])
    plsc.subcore_barrier()

    eb = wid * 80
    pltpu.sync_copy(row_hbm.at[pl.ds(eb, 80)], row_v)
    pltpu.sync_copy(col_hbm.at[pl.ds(eb, 80)], col_v)

    @pl.loop(0, 80)
    def _(j):
        pltpu.async_copy(g_hbm.at[row_v.at[j]], buf, sem).wait()
        pltpu.sync_copy(buf, part_sh.at[col_v.at[j]], add=True)

    plsc.subcore_barrier()
    pltpu.sync_copy(part_sh.at[pl.ds(nb, 656)], part_hbm.at[c, pl.ds(nb, 656)])


def kernel(x, edge_index, W_in, b_in, W_out, b_out):
    xp = jnp.zeros((NP, 128), jnp.float32).at[:NNODE].set(x)
    row = jnp.concatenate(
        [edge_index[0], jnp.zeros((EP - NEDGE,), jnp.int32)]).reshape(ECH, 128)
    col = jnp.concatenate(
        [edge_index[1], jnp.full((EP - NEDGE,), TRASH, jnp.int32)]).reshape(ECH, 128)

    h0 = _linin(xp, W_in, b_in.reshape(1, HD))
    g0, a16, inv16 = _prep_sc(col, h0)

    g = g0
    part = None
    for t in range(KSTEPS):
        part = _step_sc(g, row, col)
        if t < KSTEPS - 1:
            g = _upd(part, g, a16, g0)

    out = _fin(part, g, a16, g0, inv16, W_out, b_out.reshape(1, NCLS))
    return out[:NNODE]


# speedup vs baseline: 11.7959x; 11.7959x over previous
import dataclasses
import functools

import jax
import jax.numpy as jnp
from jax import lax
from jax.experimental import pallas as pl
from jax.experimental.pallas import tpu as pltpu
from jax.experimental.pallas import tpu_sc as plsc

NNODE = 10000
NP = 10496
TRASH = NP - 1
NEDGE = 320000
EP = 327680
ECH = EP // 128
HD = 64
NCLS = 40
KSTEPS = 4
ALPHA = 0.1

_mesh = plsc.VectorSubcoreMesh(core_axis_name="c", subcore_axis_name="s")

_sc_params = pltpu.CompilerParams()
if "needs_layout_passes" in pltpu.CompilerParams.__dataclass_fields__:
    _sc_params = dataclasses.replace(_sc_params, needs_layout_passes=False)
_sc_params = dataclasses.replace(_sc_params, use_tc_tiling_on_sc=False)


def _rsqrt16(d):
    one = jnp.full((16,), 1, jnp.int32)
    yi = jnp.full((16,), 0x5F3759DF, jnp.int32) - lax.shift_right_logical(
        plsc.bitcast(d, jnp.int32), one)
    y = plsc.bitcast(yi, jnp.float32)
    for _ in range(3):
        y = y * (1.5 - 0.5 * d * y * y)
    return y


def _linin_body(x_ref, w_ref, b_ref, o_ref):
    o_ref[...] = jnp.dot(x_ref[...], w_ref[...],
                         preferred_element_type=jnp.float32) + b_ref[...]


_linin = pl.pallas_call(
    _linin_body,
    out_shape=jax.ShapeDtypeStruct((NP, HD), jnp.float32),
)


def _upd_body(p_ref, g_ref, a_ref, g0_ref, o_ref):
    s = p_ref[0] + p_ref[1] + g_ref[...]
    o_ref[...] = a_ref[:, 0:1] * s + ALPHA * g0_ref[...]


_upd = pl.pallas_call(
    _upd_body,
    out_shape=jax.ShapeDtypeStruct((NP, HD), jnp.float32),
)


def _fin_body(g_ref, inv_ref, w_ref, b_ref, o_ref):
    h = g_ref[...] * inv_ref[:, 0:1]
    o_ref[...] = jnp.dot(jnp.maximum(h, 0.0), w_ref[...],
                         preferred_element_type=jnp.float32) + b_ref[...]


_fin = pl.pallas_call(
    _fin_body,
    out_shape=jax.ShapeDtypeStruct((NP, NCLS), jnp.float32),
)


@functools.partial(
    pl.kernel,
    out_type=[
        jax.ShapeDtypeStruct((NP, HD), jnp.float32),
        jax.ShapeDtypeStruct((NP, 16), jnp.float32),
        jax.ShapeDtypeStruct((NP, 16), jnp.float32),
    ],
    mesh=_mesh,
    scratch_types=[
        pltpu.VMEM_SHARED((NP, 16), jnp.float32),
        pltpu.VMEM((160, 128), jnp.int32),
        pltpu.VMEM((128, 16), jnp.float32),
        pltpu.VMEM((328, 16), jnp.float32),
        pltpu.VMEM((328, HD), jnp.float32),
        pltpu.VMEM((328, 16), jnp.float32),
        pltpu.VMEM((328, 16), jnp.float32),
    ],
    compiler_params=_sc_params,
)
def _prep_sc(col_hbm, h0_hbm, g0_hbm, a_hbm, inv_hbm,
             deg_sh, col_v, ones_v, deg_v, h_v, a_v, inv_v):
    s = lax.axis_index("s")
    c = lax.axis_index("c")
    wid = s * 2 + c

    one = jnp.full((16,), 1.0, jnp.float32)
    zero = jnp.zeros((16,), jnp.float32)

    @pl.loop(0, 128)
    def _(r):
        ones_v[r] = one

    @pl.loop(0, 328)
    def _(r):
        deg_v[r] = zero

    pltpu.sync_copy(deg_v, deg_sh.at[pl.ds(s * 656, 328)])
    pltpu.sync_copy(deg_v, deg_sh.at[pl.ds(s * 656 + 328, 328)])
    plsc.subcore_barrier()

    pltpu.sync_copy(col_hbm.at[pl.ds(s * 160, 160)], col_v)

    @pl.loop(0, 160)
    def _(j):
        pltpu.sync_copy(ones_v, deg_sh.at[col_v.at[j]], add=True)

    plsc.subcore_barrier()

    nb = wid * 328
    pltpu.sync_copy(deg_sh.at[pl.ds(nb, 328)], deg_v)
    pltpu.sync_copy(h0_hbm.at[pl.ds(nb, 328)], h_v)

    @pl.loop(0, 328)
    def _(r):
        d = deg_v[r] + 1.0
        y = _rsqrt16(d)
        a_v[r] = 0.9 * (y * y)
        inv_v[r] = d * y
        for f in range(4):
            h_v[r, pl.ds(f * 16, 16)] = h_v[r, pl.ds(f * 16, 16)] * y

    pltpu.sync_copy(a_v, a_hbm.at[pl.ds(nb, 328)])
    pltpu.sync_copy(inv_v, inv_hbm.at[pl.ds(nb, 328)])
    pltpu.sync_copy(h_v, g0_hbm.at[pl.ds(nb, 328)])


@functools.partial(
    pl.kernel,
    out_type=jax.ShapeDtypeStruct((2, NP, HD), jnp.float32),
    mesh=_mesh,
    scratch_types=[
        pltpu.VMEM_SHARED((NP, HD), jnp.float32),
        pltpu.VMEM((80, 128), jnp.int32),
        pltpu.VMEM((80, 128), jnp.int32),
        pltpu.VMEM((128, HD), jnp.float32),
        pltpu.SemaphoreType.DMA,
    ],
    compiler_params=_sc_params,
)
def _step_sc(g_hbm, row_hbm, col_hbm, part_hbm,
             part_sh, row_v, col_v, buf, sem):
    s = lax.axis_index("s")
    c = lax.axis_index("c")
    wid = s * 2 + c

    zero = jnp.zeros((16,), jnp.float32)

    @pl.loop(0, 128)
    def _(r):
        for f in range(4):
            buf[r, pl.ds(f * 16, 16)] = zero

    nb = s * 656

    @pl.loop(0, 5)
    def _(k):
        pltpu.sync_copy(buf, part_sh.at[pl.ds(nb + k * 128, 128)])

    pltpu.sync_copy(buf.at[pl.ds(0, 16)], part_sh.at[pl.ds(nb + 640, 16)])
    plsc.subcore_barrier()

    eb = wid * 80
    pltpu.sync_copy(row_hbm.at[pl.ds(eb, 80)], row_v)
    pltpu.sync_copy(col_hbm.at[pl.ds(eb, 80)], col_v)

    @pl.loop(0, 80)
    def _(j):
        pltpu.async_copy(g_hbm.at[row_v.at[j]], buf, sem).wait()
        pltpu.sync_copy(buf, part_sh.at[col_v.at[j]], add=True)

    plsc.subcore_barrier()
    pltpu.sync_copy(part_sh.at[pl.ds(nb, 656)], part_hbm.at[c, pl.ds(nb, 656)])


def kernel(x, edge_index, W_in, b_in, W_out, b_out):
    xp = jnp.zeros((NP, 128), jnp.float32).at[:NNODE].set(x)
    row = jnp.concatenate(
        [edge_index[0], jnp.zeros((EP - NEDGE,), jnp.int32)]).reshape(ECH, 128)
    col = jnp.concatenate(
        [edge_index[1], jnp.full((EP - NEDGE,), TRASH, jnp.int32)]).reshape(ECH, 128)

    h0 = _linin(xp, W_in, b_in.reshape(1, HD))
    g0, a16, inv16 = _prep_sc(col, h0)

    def body(g, _):
        part = _step_sc(g, row, col)
        return _upd(part, g, a16, g0), None

    g4, _ = lax.scan(body, g0, None, length=KSTEPS)

    out = _fin(g4, inv16, W_out, b_out.reshape(1, NCLS))
    return out[:NNODE]


# speedup vs baseline: 13.1205x; 1.1123x over previous
import dataclasses
import functools

import jax
import jax.numpy as jnp
from jax import lax
from jax.experimental import pallas as pl
from jax.experimental.pallas import tpu as pltpu
from jax.experimental.pallas import tpu_sc as plsc

NNODE = 10000
NP = 10496
TRASH = NP - 1
NEDGE = 320000
EP = 327680
ECH = EP // 128
HD = 64
NCLS = 40
KSTEPS = 4
ALPHA = 0.1

_mesh = plsc.VectorSubcoreMesh(core_axis_name="c", subcore_axis_name="s")

_sc_params = pltpu.CompilerParams()
if "needs_layout_passes" in pltpu.CompilerParams.__dataclass_fields__:
    _sc_params = dataclasses.replace(_sc_params, needs_layout_passes=False)
_sc_params = dataclasses.replace(_sc_params, use_tc_tiling_on_sc=False)


def _rsqrt16(d):
    one = jnp.full((16,), 1, jnp.int32)
    yi = jnp.full((16,), 0x5F3759DF, jnp.int32) - lax.shift_right_logical(
        plsc.bitcast(d, jnp.int32), one)
    y = plsc.bitcast(yi, jnp.float32)
    for _ in range(3):
        y = y * (1.5 - 0.5 * d * y * y)
    return y


def _linin_body(x_ref, w_ref, b_ref, o_ref):
    o_ref[...] = jnp.dot(x_ref[...], w_ref[...],
                         preferred_element_type=jnp.float32) + b_ref[...]


_linin = pl.pallas_call(
    _linin_body,
    out_shape=jax.ShapeDtypeStruct((NP, HD), jnp.float32),
)


def _upd_body(p_ref, g_ref, a_ref, g0_ref, o_ref):
    s = p_ref[0] + p_ref[1] + g_ref[...]
    o_ref[...] = a_ref[:, 0:1] * s + ALPHA * g0_ref[...]


_upd = pl.pallas_call(
    _upd_body,
    out_shape=jax.ShapeDtypeStruct((NP, HD), jnp.float32),
)


def _fin_body(g_ref, inv_ref, w_ref, b_ref, o_ref):
    h = g_ref[...] * inv_ref[:, 0:1]
    o_ref[...] = jnp.dot(jnp.maximum(h, 0.0), w_ref[...],
                         preferred_element_type=jnp.float32) + b_ref[...]


_fin = pl.pallas_call(
    _fin_body,
    out_shape=jax.ShapeDtypeStruct((NP, NCLS), jnp.float32),
)


@functools.partial(
    pl.kernel,
    out_type=[
        jax.ShapeDtypeStruct((NP, HD), jnp.float32),
        jax.ShapeDtypeStruct((NP, 16), jnp.float32),
        jax.ShapeDtypeStruct((NP, 16), jnp.float32),
    ],
    mesh=_mesh,
    scratch_types=[
        pltpu.VMEM_SHARED((NP, 16), jnp.float32),
        pltpu.VMEM((160, 128), jnp.int32),
        pltpu.VMEM((128, 16), jnp.float32),
        pltpu.VMEM((328, 16), jnp.float32),
        pltpu.VMEM((328, HD), jnp.float32),
        pltpu.VMEM((328, 16), jnp.float32),
        pltpu.VMEM((328, 16), jnp.float32),
    ],
    compiler_params=_sc_params,
)
def _prep_sc(col_hbm, h0_hbm, g0_hbm, a_hbm, inv_hbm,
             deg_sh, col_v, ones_v, deg_v, h_v, a_v, inv_v):
    s = lax.axis_index("s")
    c = lax.axis_index("c")
    wid = s * 2 + c

    one = jnp.full((16,), 1.0, jnp.float32)
    zero = jnp.zeros((16,), jnp.float32)

    @pl.loop(0, 128)
    def _(r):
        ones_v[r] = one

    @pl.loop(0, 328)
    def _(r):
        deg_v[r] = zero

    pltpu.sync_copy(deg_v, deg_sh.at[pl.ds(s * 656, 328)])
    pltpu.sync_copy(deg_v, deg_sh.at[pl.ds(s * 656 + 328, 328)])
    plsc.subcore_barrier()

    pltpu.sync_copy(col_hbm.at[pl.ds(s * 160, 160)], col_v)

    @pl.loop(0, 160)
    def _(j):
        pltpu.sync_copy(ones_v, deg_sh.at[col_v.at[j]], add=True)

    plsc.subcore_barrier()

    nb = wid * 328
    pltpu.sync_copy(deg_sh.at[pl.ds(nb, 328)], deg_v)
    pltpu.sync_copy(h0_hbm.at[pl.ds(nb, 328)], h_v)

    @pl.loop(0, 328)
    def _(r):
        d = deg_v[r] + 1.0
        y = _rsqrt16(d)
        a_v[r] = 0.9 * (y * y)
        inv_v[r] = d * y
        for f in range(4):
            h_v[r, pl.ds(f * 16, 16)] = h_v[r, pl.ds(f * 16, 16)] * y

    pltpu.sync_copy(a_v, a_hbm.at[pl.ds(nb, 328)])
    pltpu.sync_copy(inv_v, inv_hbm.at[pl.ds(nb, 328)])
    pltpu.sync_copy(h_v, g0_hbm.at[pl.ds(nb, 328)])


@functools.partial(
    pl.kernel,
    out_type=jax.ShapeDtypeStruct((2, NP, HD), jnp.float32),
    mesh=_mesh,
    scratch_types=[
        pltpu.VMEM_SHARED((NP, HD), jnp.float32),
        pltpu.VMEM((20, 512), jnp.int32),
        pltpu.VMEM((80, 128), jnp.int32),
        pltpu.VMEM((512, HD), jnp.float32),
        pltpu.VMEM((512, HD), jnp.float32),
        pltpu.SemaphoreType.DMA,
        pltpu.SemaphoreType.DMA,
        pltpu.SemaphoreType.DMA,
        pltpu.SemaphoreType.DMA,
    ],
    compiler_params=_sc_params,
)
def _step_sc(g_hbm, row_hbm, col_hbm, part_hbm,
             part_sh, row_v, col_v, bufa, bufb,
             sema, semb, semc, semd):
    s = lax.axis_index("s")
    c = lax.axis_index("c")
    wid = s * 2 + c

    zero = jnp.zeros((16,), jnp.float32)

    @pl.loop(0, 512)
    def _(r):
        for f in range(4):
            bufa[r, pl.ds(f * 16, 16)] = zero

    nb = s * 656
    pltpu.sync_copy(bufa, part_sh.at[pl.ds(nb, 512)])
    pltpu.sync_copy(bufa.at[pl.ds(0, 144)], part_sh.at[pl.ds(nb + 512, 144)])
    plsc.subcore_barrier()

    pltpu.sync_copy(row_hbm.at[pl.ds(wid * 20, 20)], row_v)
    pltpu.sync_copy(col_hbm.at[pl.ds(wid * 80, 80)], col_v)

    @pl.loop(0, 10)
    def _(k):
        ga = pltpu.async_copy(g_hbm.at[row_v.at[2 * k]], bufa, sema)
        gb = pltpu.async_copy(g_hbm.at[row_v.at[2 * k + 1]], bufb, semb)
        ga.wait()
        sa = [pltpu.async_copy(bufa.at[pl.ds(f * 128, 128)],
                               part_sh.at[col_v.at[8 * k + f]], semc, add=True)
              for f in range(4)]
        gb.wait()
        sb = [pltpu.async_copy(bufb.at[pl.ds(f * 128, 128)],
                               part_sh.at[col_v.at[8 * k + 4 + f]], semd, add=True)
              for f in range(4)]
        for d in sa:
            d.wait()
        for d in sb:
            d.wait()

    plsc.subcore_barrier()
    pltpu.sync_copy(part_sh.at[pl.ds(nb, 656)], part_hbm.at[c, pl.ds(nb, 656)])


def kernel(x, edge_index, W_in, b_in, W_out, b_out):
    xp = jnp.zeros((NP, 128), jnp.float32).at[:NNODE].set(x)
    row = jnp.concatenate(
        [edge_index[0], jnp.zeros((EP - NEDGE,), jnp.int32)]).reshape(ECH, 128)
    col = jnp.concatenate(
        [edge_index[1], jnp.full((EP - NEDGE,), TRASH, jnp.int32)]).reshape(ECH, 128)

    h0 = _linin(xp, W_in, b_in.reshape(1, HD))
    g0, a16, inv16 = _prep_sc(col, h0)

    row512 = row.reshape(EP // 512, 512)

    def body(g, _):
        part = _step_sc(g, row512, col)
        return _upd(part, g, a16, g0), None

    g4, _ = lax.scan(body, g0, None, length=KSTEPS)

    out = _fin(g4, inv16, W_out, b_out.reshape(1, NCLS))
    return out[:NNODE]


# speedup vs baseline: 26.0840x; 1.9880x over previous
import dataclasses
import functools

import jax
import jax.numpy as jnp
from jax import lax
from jax.experimental import pallas as pl
from jax.experimental.pallas import tpu as pltpu
from jax.experimental.pallas import tpu_sc as plsc

NNODE = 10000
NP = 10496
TRASH = NP - 1
NEDGE = 320000
EP = 327680
ECH = EP // 128
HD = 64
NCLS = 40
KSTEPS = 4
ALPHA = 0.1

_mesh = plsc.VectorSubcoreMesh(core_axis_name="c", subcore_axis_name="s")

_sc_params = pltpu.CompilerParams()
if "needs_layout_passes" in pltpu.CompilerParams.__dataclass_fields__:
    _sc_params = dataclasses.replace(_sc_params, needs_layout_passes=False)
_sc_params = dataclasses.replace(_sc_params, use_tc_tiling_on_sc=False)


def _rsqrt16(d):
    one = jnp.full((16,), 1, jnp.int32)
    yi = jnp.full((16,), 0x5F3759DF, jnp.int32) - lax.shift_right_logical(
        plsc.bitcast(d, jnp.int32), one)
    y = plsc.bitcast(yi, jnp.float32)
    for _ in range(3):
        y = y * (1.5 - 0.5 * d * y * y)
    return y


def _linin_body(x_ref, w_ref, b_ref, o_ref):
    o_ref[...] = jnp.dot(x_ref[...], w_ref[...],
                         preferred_element_type=jnp.float32) + b_ref[...]


_linin = pl.pallas_call(
    _linin_body,
    out_shape=jax.ShapeDtypeStruct((NP, HD), jnp.float32),
)


def _upd_body(p_ref, g_ref, a_ref, g0_ref, o_ref):
    s = p_ref[0] + p_ref[1] + g_ref[...]
    o_ref[...] = a_ref[:, 0:1] * s + ALPHA * g0_ref[...]


_upd = pl.pallas_call(
    _upd_body,
    out_shape=jax.ShapeDtypeStruct((NP, HD), jnp.float32),
)


def _fin_body(g_ref, inv_ref, w_ref, b_ref, o_ref):
    h = g_ref[...] * inv_ref[:, 0:1]
    o_ref[...] = jnp.dot(jnp.maximum(h, 0.0), w_ref[...],
                         preferred_element_type=jnp.float32) + b_ref[...]


_fin = pl.pallas_call(
    _fin_body,
    out_shape=jax.ShapeDtypeStruct((NP, NCLS), jnp.float32),
)


@functools.partial(
    pl.kernel,
    out_type=[
        jax.ShapeDtypeStruct((NP, HD), jnp.float32),
        jax.ShapeDtypeStruct((NP, 16), jnp.float32),
        jax.ShapeDtypeStruct((NP, 16), jnp.float32),
    ],
    mesh=_mesh,
    scratch_types=[
        pltpu.VMEM_SHARED((NP, 16), jnp.float32),
        pltpu.VMEM((160, 128), jnp.int32),
        pltpu.VMEM((128, 16), jnp.float32),
        pltpu.VMEM((328, 16), jnp.float32),
        pltpu.VMEM((328, HD), jnp.float32),
        pltpu.VMEM((328, 16), jnp.float32),
        pltpu.VMEM((328, 16), jnp.float32),
    ],
    compiler_params=_sc_params,
)
def _prep_sc(col_hbm, h0_hbm, g0_hbm, a_hbm, inv_hbm,
             deg_sh, col_v, ones_v, deg_v, h_v, a_v, inv_v):
    s = lax.axis_index("s")
    c = lax.axis_index("c")
    wid = s * 2 + c

    one = jnp.full((16,), 1.0, jnp.float32)
    zero = jnp.zeros((16,), jnp.float32)

    @pl.loop(0, 128)
    def _(r):
        ones_v[r] = one

    @pl.loop(0, 328)
    def _(r):
        deg_v[r] = zero

    pltpu.sync_copy(deg_v, deg_sh.at[pl.ds(s * 656, 328)])
    pltpu.sync_copy(deg_v, deg_sh.at[pl.ds(s * 656 + 328, 328)])
    plsc.subcore_barrier()

    pltpu.sync_copy(col_hbm.at[pl.ds(s * 160, 160)], col_v)

    @pl.loop(0, 160)
    def _(j):
        pltpu.sync_copy(ones_v, deg_sh.at[col_v.at[j]], add=True)

    plsc.subcore_barrier()

    nb = wid * 328
    pltpu.sync_copy(deg_sh.at[pl.ds(nb, 328)], deg_v)
    pltpu.sync_copy(h0_hbm.at[pl.ds(nb, 328)], h_v)

    @pl.loop(0, 328)
    def _(r):
        d = deg_v[r] + 1.0
        y = _rsqrt16(d)
        a_v[r] = 0.9 * (y * y)
        inv_v[r] = d * y
        for f in range(4):
            h_v[r, pl.ds(f * 16, 16)] = h_v[r, pl.ds(f * 16, 16)] * y

    pltpu.sync_copy(a_v, a_hbm.at[pl.ds(nb, 328)])
    pltpu.sync_copy(inv_v, inv_hbm.at[pl.ds(nb, 328)])
    pltpu.sync_copy(h_v, g0_hbm.at[pl.ds(nb, 328)])


@functools.partial(
    pl.kernel,
    out_type=jax.ShapeDtypeStruct((2, NP, HD), jnp.float32),
    mesh=_mesh,
    scratch_types=[
        pltpu.VMEM_SHARED((NP, HD), jnp.float32),
        pltpu.VMEM((20, 512), jnp.int32),
        pltpu.VMEM((80, 128), jnp.int32),
        pltpu.VMEM((512, HD), jnp.float32),
        pltpu.VMEM((512, HD), jnp.float32),
        pltpu.SemaphoreType.DMA,
        pltpu.SemaphoreType.DMA,
        pltpu.SemaphoreType.DMA,
        pltpu.SemaphoreType.DMA,
    ],
    compiler_params=_sc_params,
)
def _step_sc(g_hbm, row_hbm, col_hbm, part_hbm,
             part_sh, row_v, col_v, bufa, bufb,
             sema, semb, semc, semd):
    s = lax.axis_index("s")
    c = lax.axis_index("c")
    wid = s * 2 + c

    zero = jnp.zeros((16,), jnp.float32)

    @pl.loop(0, 512)
    def _(r):
        for f in range(4):
            bufa[r, pl.ds(f * 16, 16)] = zero

    nb = s * 656
    pltpu.sync_copy(bufa, part_sh.at[pl.ds(nb, 512)])
    pltpu.sync_copy(bufa.at[pl.ds(0, 144)], part_sh.at[pl.ds(nb + 512, 144)])
    plsc.subcore_barrier()

    pltpu.sync_copy(row_hbm.at[pl.ds(wid * 20, 20)], row_v)
    pltpu.sync_copy(col_hbm.at[pl.ds(wid * 80, 80)], col_v)

    @pl.loop(0, 10)
    def _(k):
        ga = pltpu.async_copy(g_hbm.at[row_v.at[2 * k]], bufa, sema)
        gb = pltpu.async_copy(g_hbm.at[row_v.at[2 * k + 1]], bufb, semb)
        ga.wait()
        sa = [pltpu.async_copy(bufa.at[pl.ds(f * 128, 128)],
                               part_sh.at[col_v.at[8 * k + f]], semc, add=True)
              for f in range(4)]
        gb.wait()
        sb = [pltpu.async_copy(bufb.at[pl.ds(f * 128, 128)],
                               part_sh.at[col_v.at[8 * k + 4 + f]], semd, add=True)
              for f in range(4)]
        for d in sa:
            d.wait()
        for d in sb:
            d.wait()

    plsc.subcore_barrier()
    pltpu.sync_copy(part_sh.at[pl.ds(nb, 656)], part_hbm.at[c, pl.ds(nb, 656)])


def kernel(x, edge_index, W_in, b_in, W_out, b_out):
    xp = jnp.zeros((NP, 128), jnp.float32).at[:NNODE].set(x)
    npad = EP - NEDGE
    padi = jnp.arange(npad, dtype=jnp.int32)
    row = jnp.concatenate(
        [edge_index[0], padi % NNODE]).reshape(ECH, 128)
    col = jnp.concatenate(
        [edge_index[1], NNODE + padi % (NP - NNODE)]).reshape(ECH, 128)

    h0 = _linin(xp, W_in, b_in.reshape(1, HD))
    g0, a16, inv16 = _prep_sc(col, h0)

    row512 = row.reshape(EP // 512, 512)

    def body(g, _):
        part = _step_sc(g, row512, col)
        return _upd(part, g, a16, g0), None

    g4, _ = lax.scan(body, g0, None, length=KSTEPS)

    out = _fin(g4, inv16, W_out, b_out.reshape(1, NCLS))
    return out[:NNODE]


# speedup vs baseline: 30.3796x; 1.1647x over previous
import dataclasses
import functools

import jax
import jax.numpy as jnp
from jax import lax
from jax.experimental import pallas as pl
from jax.experimental.pallas import tpu as pltpu
from jax.experimental.pallas import tpu_sc as plsc

NNODE = 10000
NP = 10496
NPT = 656
NEDGE = 320000
EP = 327680
EBT = 40
HD = 64
HH = 32
NCLS = 40
KSTEPS = 4
ALPHA = 0.1
UCH = 164

_mesh = plsc.VectorSubcoreMesh(core_axis_name="c", subcore_axis_name="s")

_sc_params = pltpu.CompilerParams()
if "needs_layout_passes" in pltpu.CompilerParams.__dataclass_fields__:
    _sc_params = dataclasses.replace(_sc_params, needs_layout_passes=False)
_sc_params = dataclasses.replace(_sc_params, use_tc_tiling_on_sc=False)


def _rsqrt16(d):
    one = jnp.full((16,), 1, jnp.int32)
    yi = jnp.full((16,), 0x5F3759DF, jnp.int32) - lax.shift_right_logical(
        plsc.bitcast(d, jnp.int32), one)
    y = plsc.bitcast(yi, jnp.float32)
    for _ in range(3):
        y = y * (1.5 - 0.5 * d * y * y)
    return y


def _linin_body(x_ref, w_ref, b_ref, o_ref):
    o_ref[...] = jnp.dot(x_ref[...], w_ref[...],
                         preferred_element_type=jnp.float32) + b_ref[...]


_linin = pl.pallas_call(
    _linin_body,
    out_shape=jax.ShapeDtypeStruct((NP, HD), jnp.float32),
)


def _fin_body(g_ref, inv_ref, w_ref, b_ref, o_ref):
    h = jnp.concatenate([g_ref[0], g_ref[1]], axis=1) * inv_ref[:, 0:1]
    o_ref[...] = jnp.dot(jnp.maximum(h, 0.0), w_ref[...],
                         preferred_element_type=jnp.float32) + b_ref[...]


_fin = pl.pallas_call(
    _fin_body,
    out_shape=jax.ShapeDtypeStruct((NP, NCLS), jnp.float32),
)


@functools.partial(
    pl.kernel,
    out_type=[
        jax.ShapeDtypeStruct((2, NP, HH), jnp.float32),
        jax.ShapeDtypeStruct((NP, 16), jnp.float32),
        jax.ShapeDtypeStruct((NP, 16), jnp.float32),
    ],
    mesh=_mesh,
    scratch_types=[
        pltpu.VMEM_SHARED((NP, 16), jnp.float32),
        pltpu.VMEM((160, 128), jnp.int32),
        pltpu.VMEM((128, 16), jnp.float32),
        pltpu.VMEM((328, 16), jnp.float32),
        pltpu.VMEM((328, HD), jnp.float32),
        pltpu.VMEM((328, HH), jnp.float32),
        pltpu.VMEM((328, HH), jnp.float32),
        pltpu.VMEM((328, 16), jnp.float32),
        pltpu.VMEM((328, 16), jnp.float32),
    ],
    compiler_params=_sc_params,
)
def _prep_sc(col_hbm, h0_hbm, g0_hbm, a_hbm, inv_hbm,
             deg_sh, col_v, ones_v, deg_v, h_v, gl_v, gr_v, a_v, inv_v):
    s = lax.axis_index("s")
    c = lax.axis_index("c")
    wid = s * 2 + c

    one = jnp.full((16,), 1.0, jnp.float32)
    zero = jnp.zeros((16,), jnp.float32)

    @pl.loop(0, 128)
    def _(r):
        ones_v[r] = one

    @pl.loop(0, 328)
    def _(r):
        deg_v[r] = zero

    pltpu.sync_copy(deg_v, deg_sh.at[pl.ds(s * 656, 328)])
    pltpu.sync_copy(deg_v, deg_sh.at[pl.ds(s * 656 + 328, 328)])
    plsc.subcore_barrier()

    pltpu.sync_copy(col_hbm.at[pl.ds(s * 160, 160)], col_v)

    @pl.loop(0, 160)
    def _(j):
        pltpu.sync_copy(ones_v, deg_sh.at[col_v.at[j]], add=True)

    plsc.subcore_barrier()

    nb = wid * 328
    pltpu.sync_copy(deg_sh.at[pl.ds(nb, 328)], deg_v)
    pltpu.sync_copy(h0_hbm.at[pl.ds(nb, 328)], h_v)

    @pl.loop(0, 328)
    def _(r):
        d = deg_v[r] + 1.0
        y = _rsqrt16(d)
        a_v[r] = 0.9 * (y * y)
        inv_v[r] = d * y
        for f in range(2):
            gl_v[r, pl.ds(f * 16, 16)] = h_v[r, pl.ds(f * 16, 16)] * y
            gr_v[r, pl.ds(f * 16, 16)] = h_v[r, pl.ds(32 + f * 16, 16)] * y

    pltpu.sync_copy(a_v, a_hbm.at[pl.ds(nb, 328)])
    pltpu.sync_copy(inv_v, inv_hbm.at[pl.ds(nb, 328)])
    pltpu.sync_copy(gl_v, g0_hbm.at[0, pl.ds(nb, 328)])
    pltpu.sync_copy(gr_v, g0_hbm.at[1, pl.ds(nb, 328)])


@functools.partial(
    pl.kernel,
    out_type=jax.ShapeDtypeStruct((2, NP, HH), jnp.float32),
    mesh=_mesh,
    scratch_types=[
        pltpu.VMEM_SHARED((NP, HH), jnp.float32),
        pltpu.VMEM((EBT, 512), jnp.int32),
        pltpu.VMEM((4 * EBT, 128), jnp.int32),
        pltpu.VMEM((512, HH), jnp.float32),
        pltpu.VMEM((512, HH), jnp.float32),
        pltpu.VMEM((UCH, HH), jnp.float32),
        pltpu.VMEM((UCH, HH), jnp.float32),
        pltpu.VMEM((UCH, HH), jnp.float32),
        pltpu.VMEM((UCH, 16), jnp.float32),
        pltpu.SemaphoreType.DMA,
        pltpu.SemaphoreType.DMA,
        pltpu.SemaphoreType.DMA,
        pltpu.SemaphoreType.DMA,
    ],
    compiler_params=_sc_params,
)
def _appnp_sc(g0_hbm, a_hbm, row_hbm, col_hbm, g_hbm,
              s_sh, row_v, col_v, bufa, bufb, sv, gv, g0v, av,
              sema, semb, semc, semd):
    s = lax.axis_index("s")
    c = lax.axis_index("c")
    nb = s * NPT

    zero = jnp.zeros((16,), jnp.float32)

    pltpu.sync_copy(row_hbm.at[pl.ds(s * EBT, EBT)], row_v)
    pltpu.sync_copy(col_hbm.at[pl.ds(s * 4 * EBT, 4 * EBT)], col_v)

    @pl.loop(0, UCH)
    def _(r):
        for f in range(2):
            sv[r, pl.ds(f * 16, 16)] = zero

    for u in range(4):
        pltpu.sync_copy(g0_hbm.at[c, pl.ds(nb + u * UCH, UCH)], gv)
        pltpu.sync_copy(gv, g_hbm.at[c, pl.ds(nb + u * UCH, UCH)])
        pltpu.sync_copy(sv, s_sh.at[pl.ds(nb + u * UCH, UCH)])
    plsc.subcore_barrier()

    @pl.loop(0, KSTEPS)
    def _(t):
        @pl.loop(0, EBT // 2)
        def _(k):
            ga = pltpu.async_copy(g_hbm.at[c].at[row_v.at[2 * k]], bufa, sema)
            gb = pltpu.async_copy(g_hbm.at[c].at[row_v.at[2 * k + 1]], bufb, semb)
            ga.wait()
            sa = [pltpu.async_copy(bufa.at[pl.ds(f * 128, 128)],
                                   s_sh.at[col_v.at[8 * k + f]], semc, add=True)
                  for f in range(4)]
            gb.wait()
            sb = [pltpu.async_copy(bufb.at[pl.ds(f * 128, 128)],
                                   s_sh.at[col_v.at[8 * k + 4 + f]], semd, add=True)
                  for f in range(4)]
            for dsc in sa:
                dsc.wait()
            for dsc in sb:
                dsc.wait()

        plsc.subcore_barrier()

        for u in range(4):
            rb = nb + u * UCH
            pltpu.sync_copy(s_sh.at[pl.ds(rb, UCH)], sv)
            pltpu.sync_copy(g_hbm.at[c, pl.ds(rb, UCH)], gv)
            pltpu.sync_copy(g0_hbm.at[c, pl.ds(rb, UCH)], g0v)
            pltpu.sync_copy(a_hbm.at[pl.ds(rb, UCH)], av)

            @pl.loop(0, UCH)
            def _(r):
                a = av[r]
                for f in range(2):
                    fsl = pl.ds(f * 16, 16)
                    gv[r, fsl] = (a * (sv[r, fsl] + gv[r, fsl])
                                  + ALPHA * g0v[r, fsl])
                    sv[r, fsl] = zero

            pltpu.sync_copy(gv, g_hbm.at[c, pl.ds(rb, UCH)])
            pltpu.sync_copy(sv, s_sh.at[pl.ds(rb, UCH)])

        plsc.subcore_barrier()


def kernel(x, edge_index, W_in, b_in, W_out, b_out):
    xp = jnp.zeros((NP, 128), jnp.float32).at[:NNODE].set(x)
    npad = EP - NEDGE
    padi = jnp.arange(npad, dtype=jnp.int32)
    row = jnp.concatenate([edge_index[0], padi % NNODE]).reshape(EP // 512, 512)
    col = jnp.concatenate(
        [edge_index[1], NNODE + padi % (NP - NNODE)]).reshape(EP // 128, 128)

    h0 = _linin(xp, W_in, b_in.reshape(1, HD))
    g0h, a16, inv16 = _prep_sc(col, h0)
    g4h = _appnp_sc(g0h, a16, row, col)
    out = _fin(g4h, inv16, W_out, b_out.reshape(1, NCLS))
    return out[:NNODE]


# speedup vs baseline: 30.8680x; 1.0161x over previous
import dataclasses
import functools

import jax
import jax.numpy as jnp
from jax import lax
from jax.experimental import pallas as pl
from jax.experimental.pallas import tpu as pltpu
from jax.experimental.pallas import tpu_sc as plsc

NNODE = 10000
NP = 10496
NPT = 656
NEDGE = 320000
EP = 327680
EBT = 40
HD = 64
HH = 32
NCLS = 40
KSTEPS = 4
ALPHA = 0.1
UCH = 164

_mesh = plsc.VectorSubcoreMesh(core_axis_name="c", subcore_axis_name="s")

_sc_params = pltpu.CompilerParams()
if "needs_layout_passes" in pltpu.CompilerParams.__dataclass_fields__:
    _sc_params = dataclasses.replace(_sc_params, needs_layout_passes=False)
_sc_params = dataclasses.replace(_sc_params, use_tc_tiling_on_sc=False)


def _rsqrt16(d):
    one = jnp.full((16,), 1, jnp.int32)
    yi = jnp.full((16,), 0x5F3759DF, jnp.int32) - lax.shift_right_logical(
        plsc.bitcast(d, jnp.int32), one)
    y = plsc.bitcast(yi, jnp.float32)
    for _ in range(3):
        y = y * (1.5 - 0.5 * d * y * y)
    return y


def _linin_body(x_ref, w_ref, b_ref, o_ref):
    o_ref[...] = jnp.dot(x_ref[...], w_ref[...],
                         preferred_element_type=jnp.float32) + b_ref[...]


_linin = pl.pallas_call(
    _linin_body,
    out_shape=jax.ShapeDtypeStruct((NP, HD), jnp.float32),
)


def _fin_body(g_ref, inv_ref, w_ref, b_ref, o_ref):
    h = jnp.concatenate([g_ref[0], g_ref[1]], axis=1) * inv_ref[:, 0:1]
    o_ref[...] = jnp.dot(jnp.maximum(h, 0.0), w_ref[...],
                         preferred_element_type=jnp.float32) + b_ref[...]


_fin = pl.pallas_call(
    _fin_body,
    out_shape=jax.ShapeDtypeStruct((NP, NCLS), jnp.float32),
)


@functools.partial(
    pl.kernel,
    out_type=[
        jax.ShapeDtypeStruct((2, NP, HH), jnp.float32),
        jax.ShapeDtypeStruct((NP, 16), jnp.float32),
        jax.ShapeDtypeStruct((NP, 16), jnp.float32),
    ],
    mesh=_mesh,
    scratch_types=[
        pltpu.VMEM_SHARED((NP, 16), jnp.float32),
        pltpu.VMEM((40, 512), jnp.int32),
        pltpu.VMEM((512, 16), jnp.float32),
        pltpu.VMEM((328, 16), jnp.float32),
        pltpu.VMEM((328, HD), jnp.float32),
        pltpu.VMEM((328, HH), jnp.float32),
        pltpu.VMEM((328, HH), jnp.float32),
        pltpu.VMEM((328, 16), jnp.float32),
        pltpu.VMEM((328, 16), jnp.float32),
    ],
    compiler_params=_sc_params,
)
def _prep_sc(col_hbm, h0_hbm, g0_hbm, a_hbm, inv_hbm,
             deg_sh, col_v, ones_v, deg_v, h_v, gl_v, gr_v, a_v, inv_v):
    s = lax.axis_index("s")
    c = lax.axis_index("c")
    wid = s * 2 + c

    one = jnp.full((16,), 1.0, jnp.float32)
    zero = jnp.zeros((16,), jnp.float32)

    @pl.loop(0, 512)
    def _(r):
        ones_v[r] = one

    @pl.loop(0, 328)
    def _(r):
        deg_v[r] = zero

    pltpu.sync_copy(deg_v, deg_sh.at[pl.ds(s * 656, 328)])
    pltpu.sync_copy(deg_v, deg_sh.at[pl.ds(s * 656 + 328, 328)])
    plsc.subcore_barrier()

    pltpu.sync_copy(col_hbm.at[pl.ds(s * 40, 40)], col_v)

    @pl.loop(0, 40)
    def _(j):
        pltpu.sync_copy(ones_v, deg_sh.at[col_v.at[j]], add=True)

    plsc.subcore_barrier()

    nb = wid * 328
    pltpu.sync_copy(deg_sh.at[pl.ds(nb, 328)], deg_v)
    pltpu.sync_copy(h0_hbm.at[pl.ds(nb, 328)], h_v)

    @pl.loop(0, 328)
    def _(r):
        d = deg_v[r] + 1.0
        y = _rsqrt16(d)
        a_v[r] = 0.9 * (y * y)
        inv_v[r] = d * y
        for f in range(2):
            gl_v[r, pl.ds(f * 16, 16)] = h_v[r, pl.ds(f * 16, 16)] * y
            gr_v[r, pl.ds(f * 16, 16)] = h_v[r, pl.ds(32 + f * 16, 16)] * y

    pltpu.sync_copy(a_v, a_hbm.at[pl.ds(nb, 328)])
    pltpu.sync_copy(inv_v, inv_hbm.at[pl.ds(nb, 328)])
    pltpu.sync_copy(gl_v, g0_hbm.at[0, pl.ds(nb, 328)])
    pltpu.sync_copy(gr_v, g0_hbm.at[1, pl.ds(nb, 328)])


@functools.partial(
    pl.kernel,
    out_type=jax.ShapeDtypeStruct((2, NP, HH), jnp.float32),
    mesh=_mesh,
    scratch_types=[
        pltpu.VMEM_SHARED((NP, HH), jnp.float32),
        pltpu.VMEM((EBT, 512), jnp.int32),
        pltpu.VMEM((EBT, 512), jnp.int32),
        pltpu.VMEM((512, HH), jnp.float32),
        pltpu.VMEM((512, HH), jnp.float32),
        pltpu.VMEM((UCH, HH), jnp.float32),
        pltpu.VMEM((UCH, HH), jnp.float32),
        pltpu.VMEM((UCH, HH), jnp.float32),
        pltpu.VMEM((UCH, 16), jnp.float32),
        pltpu.SemaphoreType.DMA,
        pltpu.SemaphoreType.DMA,
        pltpu.SemaphoreType.DMA,
        pltpu.SemaphoreType.DMA,
    ],
    compiler_params=_sc_params,
)
def _appnp_sc(g0_hbm, a_hbm, row_hbm, col_hbm, g_hbm,
              s_sh, row_v, col_v, bufa, bufb, sv, gv, g0v, av,
              sema, semb, semc, semd):
    s = lax.axis_index("s")
    c = lax.axis_index("c")
    nb = s * NPT

    zero = jnp.zeros((16,), jnp.float32)

    pltpu.sync_copy(row_hbm.at[pl.ds(s * EBT, EBT)], row_v)
    pltpu.sync_copy(col_hbm.at[pl.ds(s * EBT, EBT)], col_v)

    @pl.loop(0, UCH)
    def _(r):
        for f in range(2):
            sv[r, pl.ds(f * 16, 16)] = zero

    for u in range(4):
        pltpu.sync_copy(g0_hbm.at[c, pl.ds(nb + u * UCH, UCH)], gv)
        pltpu.sync_copy(gv, g_hbm.at[c, pl.ds(nb + u * UCH, UCH)])
        pltpu.sync_copy(sv, s_sh.at[pl.ds(nb + u * UCH, UCH)])
    plsc.subcore_barrier()

    @pl.loop(0, KSTEPS)
    def _(t):
        @pl.loop(0, EBT // 2)
        def _(k):
            ga = pltpu.async_copy(g_hbm.at[c].at[row_v.at[2 * k]], bufa, sema)
            gb = pltpu.async_copy(g_hbm.at[c].at[row_v.at[2 * k + 1]], bufb, semb)
            ga.wait()
            sa = pltpu.async_copy(bufa, s_sh.at[col_v.at[2 * k]], semc, add=True)
            gb.wait()
            sb = pltpu.async_copy(bufb, s_sh.at[col_v.at[2 * k + 1]], semd,
                                  add=True)
            sa.wait()
            sb.wait()

        plsc.subcore_barrier()

        for u in range(4):
            rb = nb + u * UCH
            pltpu.sync_copy(s_sh.at[pl.ds(rb, UCH)], sv)
            pltpu.sync_copy(g_hbm.at[c, pl.ds(rb, UCH)], gv)
            pltpu.sync_copy(g0_hbm.at[c, pl.ds(rb, UCH)], g0v)
            pltpu.sync_copy(a_hbm.at[pl.ds(rb, UCH)], av)

            @pl.loop(0, UCH)
            def _(r):
                a = av[r]
                for f in range(2):
                    fsl = pl.ds(f * 16, 16)
                    gv[r, fsl] = (a * (sv[r, fsl] + gv[r, fsl])
                                  + ALPHA * g0v[r, fsl])
                    sv[r, fsl] = zero

            pltpu.sync_copy(gv, g_hbm.at[c, pl.ds(rb, UCH)])
            pltpu.sync_copy(sv, s_sh.at[pl.ds(rb, UCH)])

        plsc.subcore_barrier()


def kernel(x, edge_index, W_in, b_in, W_out, b_out):
    xp = jnp.zeros((NP, 128), jnp.float32).at[:NNODE].set(x)
    npad = EP - NEDGE
    padi = jnp.arange(npad, dtype=jnp.int32)
    row = jnp.concatenate([edge_index[0], padi % NNODE]).reshape(EP // 512, 512)
    col = jnp.concatenate(
        [edge_index[1], NNODE + padi % (NP - NNODE)]).reshape(EP // 512, 512)

    h0 = _linin(xp, W_in, b_in.reshape(1, HD))
    g0h, a16, inv16 = _prep_sc(col, h0)
    g4h = _appnp_sc(g0h, a16, row, col)
    out = _fin(g4h, inv16, W_out, b_out.reshape(1, NCLS))
    return out[:NNODE]


# speedup vs baseline: 34.9084x; 1.1309x over previous
import dataclasses
import functools

import jax
import jax.numpy as jnp
from jax import lax
from jax.experimental import pallas as pl
from jax.experimental.pallas import tpu as pltpu
from jax.experimental.pallas import tpu_sc as plsc

NNODE = 10000
NP = 10496
NPT = 656
NEDGE = 320000
EP = 327680
EBT = 40
HD = 64
HH = 32
NCLS = 40
KSTEPS = 4
ALPHA = 0.1
UCH = 164

_mesh = plsc.VectorSubcoreMesh(core_axis_name="c", subcore_axis_name="s")

_sc_params = pltpu.CompilerParams()
if "needs_layout_passes" in pltpu.CompilerParams.__dataclass_fields__:
    _sc_params = dataclasses.replace(_sc_params, needs_layout_passes=False)
_sc_params = dataclasses.replace(_sc_params, use_tc_tiling_on_sc=False)


def _rsqrt16(d):
    one = jnp.full((16,), 1, jnp.int32)
    yi = jnp.full((16,), 0x5F3759DF, jnp.int32) - lax.shift_right_logical(
        plsc.bitcast(d, jnp.int32), one)
    y = plsc.bitcast(yi, jnp.float32)
    for _ in range(3):
        y = y * (1.5 - 0.5 * d * y * y)
    return y


def _linin_body(x_ref, w_ref, b_ref, o_ref):
    o_ref[...] = jnp.dot(x_ref[...], w_ref[...],
                         preferred_element_type=jnp.float32) + b_ref[...]


_linin = pl.pallas_call(
    _linin_body,
    out_shape=jax.ShapeDtypeStruct((NP, HD), jnp.float32),
)


def _fin_body(g_ref, inv_ref, w_ref, b_ref, o_ref):
    h = jnp.concatenate([g_ref[0], g_ref[1]], axis=1) * inv_ref[:, 0:1]
    o_ref[...] = jnp.dot(jnp.maximum(h, 0.0), w_ref[...],
                         preferred_element_type=jnp.float32) + b_ref[...]


_fin = pl.pallas_call(
    _fin_body,
    out_shape=jax.ShapeDtypeStruct((NP, NCLS), jnp.float32),
)


@functools.partial(
    pl.kernel,
    out_type=[
        jax.ShapeDtypeStruct((2, NP, HH), jnp.float32),
        jax.ShapeDtypeStruct((NP, 16), jnp.float32),
        jax.ShapeDtypeStruct((NP, 16), jnp.float32),
    ],
    mesh=_mesh,
    scratch_types=[
        pltpu.VMEM_SHARED((NP, 16), jnp.float32),
        pltpu.VMEM((40, 512), jnp.int32),
        pltpu.VMEM((512, 16), jnp.float32),
        pltpu.VMEM((328, 16), jnp.float32),
        pltpu.VMEM((328, HD), jnp.float32),
        pltpu.VMEM((328, HH), jnp.float32),
        pltpu.VMEM((328, HH), jnp.float32),
        pltpu.VMEM((328, 16), jnp.float32),
        pltpu.VMEM((328, 16), jnp.float32),
    ],
    compiler_params=_sc_params,
)
def _prep_sc(col_hbm, h0_hbm, g0_hbm, a_hbm, inv_hbm,
             deg_sh, col_v, ones_v, deg_v, h_v, gl_v, gr_v, a_v, inv_v):
    s = lax.axis_index("s")
    c = lax.axis_index("c")
    wid = s * 2 + c

    one = jnp.full((16,), 1.0, jnp.float32)
    zero = jnp.zeros((16,), jnp.float32)

    @pl.loop(0, 512)
    def _(r):
        ones_v[r] = one

    @pl.loop(0, 328)
    def _(r):
        deg_v[r] = zero

    pltpu.sync_copy(deg_v, deg_sh.at[pl.ds(s * 656, 328)])
    pltpu.sync_copy(deg_v, deg_sh.at[pl.ds(s * 656 + 328, 328)])
    plsc.subcore_barrier()

    pltpu.sync_copy(col_hbm.at[pl.ds(s * 40, 40)], col_v)

    @pl.loop(0, 40)
    def _(j):
        pltpu.sync_copy(ones_v, deg_sh.at[col_v.at[j]], add=True)

    plsc.subcore_barrier()

    nb = wid * 328
    pltpu.sync_copy(deg_sh.at[pl.ds(nb, 328)], deg_v)
    pltpu.sync_copy(h0_hbm.at[pl.ds(nb, 328)], h_v)

    @pl.loop(0, 328)
    def _(r):
        d = deg_v[r] + 1.0
        y = _rsqrt16(d)
        a_v[r] = 0.9 * (y * y)
        inv_v[r] = d * y
        for f in range(2):
            gl_v[r, pl.ds(f * 16, 16)] = h_v[r, pl.ds(f * 16, 16)] * y
            gr_v[r, pl.ds(f * 16, 16)] = h_v[r, pl.ds(32 + f * 16, 16)] * y

    pltpu.sync_copy(a_v, a_hbm.at[pl.ds(nb, 328)])
    pltpu.sync_copy(inv_v, inv_hbm.at[pl.ds(nb, 328)])
    pltpu.sync_copy(gl_v, g0_hbm.at[0, pl.ds(nb, 328)])
    pltpu.sync_copy(gr_v, g0_hbm.at[1, pl.ds(nb, 328)])


@functools.partial(
    pl.kernel,
    out_type=jax.ShapeDtypeStruct((2, NP, HH), jnp.float32),
    mesh=_mesh,
    scratch_types=[
        pltpu.VMEM_SHARED((NP, HH), jnp.float32),
        pltpu.VMEM((EBT // 2, 512), jnp.int32),
        pltpu.VMEM((EBT // 2, 512), jnp.int32),
        pltpu.VMEM((512, HH), jnp.float32),
        pltpu.VMEM((512, HH), jnp.float32),
        pltpu.VMEM((512, HH), jnp.float32),
        pltpu.VMEM((512, HH), jnp.float32),
        pltpu.VMEM((UCH, HH), jnp.float32),
        pltpu.VMEM((UCH, HH), jnp.float32),
        pltpu.VMEM((UCH, HH), jnp.float32),
        pltpu.VMEM((UCH, 16), jnp.float32),
        pltpu.SemaphoreType.DMA,
        pltpu.SemaphoreType.DMA,
        pltpu.SemaphoreType.DMA,
        pltpu.SemaphoreType.DMA,
        pltpu.SemaphoreType.DMA,
        pltpu.SemaphoreType.DMA,
        pltpu.SemaphoreType.DMA,
        pltpu.SemaphoreType.DMA,
    ],
    compiler_params=_sc_params,
)
def _appnp_sc(g0_hbm, a_hbm, row_hbm, col_hbm, g_hbm,
              s_sh, row_v, col_v, bufa, bufb, bufc, bufd, sv, gv, g0v, av,
              sga, sgb, sgc, sgd, ssa, ssb, ssc, ssd):
    s = lax.axis_index("s")
    c = lax.axis_index("c")
    nb = s * NPT

    zero = jnp.zeros((16,), jnp.float32)

    @pl.loop(0, UCH)
    def _(r):
        for f in range(2):
            sv[r, pl.ds(f * 16, 16)] = zero

    for u in range(4):
        pltpu.sync_copy(g0_hbm.at[c, pl.ds(nb + u * UCH, UCH)], gv)
        pltpu.sync_copy(gv, g_hbm.at[c, pl.ds(nb + u * UCH, UCH)])
        pltpu.sync_copy(sv, s_sh.at[pl.ds(nb + u * UCH, UCH)])
    plsc.subcore_barrier()

    ghalf = g_hbm.at[c]

    @pl.loop(0, KSTEPS)
    def _(t):
        for h in range(2):
            hb = s * EBT + h * (EBT // 2)
            pltpu.sync_copy(row_hbm.at[pl.ds(hb, EBT // 2)], row_v)
            pltpu.sync_copy(col_hbm.at[pl.ds(hb, EBT // 2)], col_v)
            pltpu.async_copy(ghalf.at[row_v.at[0]], bufa, sga)
            pltpu.async_copy(ghalf.at[row_v.at[1]], bufb, sgb)

            @pl.loop(0, 5)
            def _(j):
                b = 4 * j

                @pl.when(j > 0)
                def _():
                    pltpu.make_async_copy(bufc, s_sh.at[col_v.at[0]], ssc).wait()
                    pltpu.make_async_copy(bufd, s_sh.at[col_v.at[0]], ssd).wait()

                pltpu.async_copy(ghalf.at[row_v.at[b + 2]], bufc, sgc)
                pltpu.async_copy(ghalf.at[row_v.at[b + 3]], bufd, sgd)
                pltpu.make_async_copy(ghalf.at[row_v.at[0]], bufa, sga).wait()
                sa = pltpu.async_copy(bufa, s_sh.at[col_v.at[b]], ssa, add=True)
                pltpu.make_async_copy(ghalf.at[row_v.at[0]], bufb, sgb).wait()
                sb = pltpu.async_copy(bufb, s_sh.at[col_v.at[b + 1]], ssb,
                                      add=True)

                @pl.when(j < 4)
                def _():
                    sa.wait()
                    sb.wait()
                    pltpu.async_copy(ghalf.at[row_v.at[b + 4]], bufa, sga)
                    pltpu.async_copy(ghalf.at[row_v.at[b + 5]], bufb, sgb)

                pltpu.make_async_copy(ghalf.at[row_v.at[0]], bufc, sgc).wait()
                pltpu.async_copy(bufc, s_sh.at[col_v.at[b + 2]], ssc, add=True)
                pltpu.make_async_copy(ghalf.at[row_v.at[0]], bufd, sgd).wait()
                pltpu.async_copy(bufd, s_sh.at[col_v.at[b + 3]], ssd, add=True)

            pltpu.make_async_copy(bufa, s_sh.at[col_v.at[0]], ssa).wait()
            pltpu.make_async_copy(bufb, s_sh.at[col_v.at[0]], ssb).wait()
            pltpu.make_async_copy(bufc, s_sh.at[col_v.at[0]], ssc).wait()
            pltpu.make_async_copy(bufd, s_sh.at[col_v.at[0]], ssd).wait()

        plsc.subcore_barrier()

        for u in range(4):
            rb = nb + u * UCH
            pltpu.sync_copy(s_sh.at[pl.ds(rb, UCH)], sv)
            pltpu.sync_copy(g_hbm.at[c, pl.ds(rb, UCH)], gv)
            pltpu.sync_copy(g0_hbm.at[c, pl.ds(rb, UCH)], g0v)
            pltpu.sync_copy(a_hbm.at[pl.ds(rb, UCH)], av)

            @pl.loop(0, UCH)
            def _(r):
                a = av[r]
                for f in range(2):
                    fsl = pl.ds(f * 16, 16)
                    gv[r, fsl] = (a * (sv[r, fsl] + gv[r, fsl])
                                  + ALPHA * g0v[r, fsl])
                    sv[r, fsl] = zero

            pltpu.sync_copy(gv, g_hbm.at[c, pl.ds(rb, UCH)])
            pltpu.sync_copy(sv, s_sh.at[pl.ds(rb, UCH)])

        plsc.subcore_barrier()


def kernel(x, edge_index, W_in, b_in, W_out, b_out):
    xp = jnp.zeros((NP, 128), jnp.float32).at[:NNODE].set(x)
    npad = EP - NEDGE
    padi = jnp.arange(npad, dtype=jnp.int32)
    row = jnp.concatenate([edge_index[0], padi % NNODE]).reshape(EP // 512, 512)
    col = jnp.concatenate(
        [edge_index[1], NNODE + padi % (NP - NNODE)]).reshape(EP // 512, 512)

    h0 = _linin(xp, W_in, b_in.reshape(1, HD))
    g0h, a16, inv16 = _prep_sc(col, h0)
    g4h = _appnp_sc(g0h, a16, row, col)
    out = _fin(g4h, inv16, W_out, b_out.reshape(1, NCLS))
    return out[:NNODE]


# speedup vs baseline: 35.1189x; 1.0060x over previous
import dataclasses
import functools

import jax
import jax.numpy as jnp
from jax import lax
from jax.experimental import pallas as pl
from jax.experimental.pallas import tpu as pltpu
from jax.experimental.pallas import tpu_sc as plsc

NNODE = 10000
NP = 10496
NPT = 656
NEDGE = 320000
EP = 327680
EBT = 40
HD = 64
HH = 32
NCLS = 40
KSTEPS = 4
ALPHA = 0.1
UCH = 164

_mesh = plsc.VectorSubcoreMesh(core_axis_name="c", subcore_axis_name="s")

_sc_params = pltpu.CompilerParams()
if "needs_layout_passes" in pltpu.CompilerParams.__dataclass_fields__:
    _sc_params = dataclasses.replace(_sc_params, needs_layout_passes=False)
_sc_params = dataclasses.replace(_sc_params, use_tc_tiling_on_sc=False)


def _rsqrt16(d):
    one = jnp.full((16,), 1, jnp.int32)
    yi = jnp.full((16,), 0x5F3759DF, jnp.int32) - lax.shift_right_logical(
        plsc.bitcast(d, jnp.int32), one)
    y = plsc.bitcast(yi, jnp.float32)
    for _ in range(3):
        y = y * (1.5 - 0.5 * d * y * y)
    return y


def _linin_body(x_ref, w_ref, b_ref, o_ref):
    o_ref[...] = jnp.broadcast_to(b_ref[...], (NP, HD))
    o_ref[0:NNODE] = jnp.dot(x_ref[...], w_ref[...],
                             preferred_element_type=jnp.float32) + b_ref[...]


_linin = pl.pallas_call(
    _linin_body,
    out_shape=jax.ShapeDtypeStruct((NP, HD), jnp.float32),
)


def _fin_body(g_ref, inv_ref, w_ref, b_ref, o_ref):
    h = jnp.concatenate([g_ref[0], g_ref[1]], axis=1) * inv_ref[:, 0:1]
    o_ref[...] = jnp.dot(jnp.maximum(h, 0.0), w_ref[...],
                         preferred_element_type=jnp.float32) + b_ref[...]


_fin = pl.pallas_call(
    _fin_body,
    out_shape=jax.ShapeDtypeStruct((NP, NCLS), jnp.float32),
)


@functools.partial(
    pl.kernel,
    out_type=[
        jax.ShapeDtypeStruct((2, NP, HH), jnp.float32),
        jax.ShapeDtypeStruct((NP, 16), jnp.float32),
        jax.ShapeDtypeStruct((NP, 16), jnp.float32),
    ],
    mesh=_mesh,
    scratch_types=[
        pltpu.VMEM_SHARED((NP, 16), jnp.float32),
        pltpu.VMEM((40, 512), jnp.int32),
        pltpu.VMEM((512, 16), jnp.float32),
        pltpu.VMEM((328, 16), jnp.float32),
        pltpu.VMEM((328, HD), jnp.float32),
        pltpu.VMEM((328, HH), jnp.float32),
        pltpu.VMEM((328, HH), jnp.float32),
        pltpu.VMEM((328, 16), jnp.float32),
        pltpu.VMEM((328, 16), jnp.float32),
    ],
    compiler_params=_sc_params,
)
def _prep_sc(col_hbm, h0_hbm, g0_hbm, a_hbm, inv_hbm,
             deg_sh, col_v, ones_v, deg_v, h_v, gl_v, gr_v, a_v, inv_v):
    s = lax.axis_index("s")
    c = lax.axis_index("c")
    wid = s * 2 + c

    one = jnp.full((16,), 1.0, jnp.float32)
    zero = jnp.zeros((16,), jnp.float32)

    @pl.loop(0, 512)
    def _(r):
        ones_v[r] = one

    @pl.loop(0, 328)
    def _(r):
        deg_v[r] = zero

    pltpu.sync_copy(deg_v, deg_sh.at[pl.ds(s * 656, 328)])
    pltpu.sync_copy(deg_v, deg_sh.at[pl.ds(s * 656 + 328, 328)])
    plsc.subcore_barrier()

    pltpu.sync_copy(col_hbm.at[pl.ds(s * 40, 40)], col_v)

    @pl.loop(0, 40)
    def _(j):
        pltpu.sync_copy(ones_v, deg_sh.at[col_v.at[j]], add=True)

    plsc.subcore_barrier()

    nb = wid * 328
    pltpu.sync_copy(deg_sh.at[pl.ds(nb, 328)], deg_v)
    pltpu.sync_copy(h0_hbm.at[pl.ds(nb, 328)], h_v)

    @pl.loop(0, 328)
    def _(r):
        d = deg_v[r] + 1.0
        y = _rsqrt16(d)
        a_v[r] = 0.9 * (y * y)
        inv_v[r] = d * y
        for f in range(2):
            gl_v[r, pl.ds(f * 16, 16)] = h_v[r, pl.ds(f * 16, 16)] * y
            gr_v[r, pl.ds(f * 16, 16)] = h_v[r, pl.ds(32 + f * 16, 16)] * y

    pltpu.sync_copy(a_v, a_hbm.at[pl.ds(nb, 328)])
    pltpu.sync_copy(inv_v, inv_hbm.at[pl.ds(nb, 328)])
    pltpu.sync_copy(gl_v, g0_hbm.at[0, pl.ds(nb, 328)])
    pltpu.sync_copy(gr_v, g0_hbm.at[1, pl.ds(nb, 328)])


@functools.partial(
    pl.kernel,
    out_type=jax.ShapeDtypeStruct((2, NP, HH), jnp.float32),
    mesh=_mesh,
    scratch_types=[
        pltpu.VMEM_SHARED((NP, HH), jnp.float32),
        pltpu.VMEM((EBT // 2, 512), jnp.int32),
        pltpu.VMEM((EBT // 2, 512), jnp.int32),
        pltpu.VMEM((512, HH), jnp.float32),
        pltpu.VMEM((512, HH), jnp.float32),
        pltpu.VMEM((512, HH), jnp.float32),
        pltpu.VMEM((512, HH), jnp.float32),
        pltpu.VMEM((UCH, HH), jnp.float32),
        pltpu.VMEM((UCH, HH), jnp.float32),
        pltpu.VMEM((UCH, HH), jnp.float32),
        pltpu.VMEM((UCH, 16), jnp.float32),
        pltpu.SemaphoreType.DMA,
        pltpu.SemaphoreType.DMA,
        pltpu.SemaphoreType.DMA,
        pltpu.SemaphoreType.DMA,
        pltpu.SemaphoreType.DMA,
        pltpu.SemaphoreType.DMA,
        pltpu.SemaphoreType.DMA,
        pltpu.SemaphoreType.DMA,
    ],
    compiler_params=_sc_params,
)
def _appnp_sc(g0_hbm, a_hbm, row_hbm, col_hbm, g_hbm,
              s_sh, row_v, col_v, bufa, bufb, bufc, bufd, sv, gv, g0v, av,
              sga, sgb, sgc, sgd, ssa, ssb, ssc, ssd):
    s = lax.axis_index("s")
    c = lax.axis_index("c")
    nb = s * NPT

    zero = jnp.zeros((16,), jnp.float32)

    @pl.loop(0, UCH)
    def _(r):
        for f in range(2):
            sv[r, pl.ds(f * 16, 16)] = zero

    for u in range(4):
        pltpu.sync_copy(g0_hbm.at[c, pl.ds(nb + u * UCH, UCH)], gv)
        pltpu.sync_copy(gv, g_hbm.at[c, pl.ds(nb + u * UCH, UCH)])
        pltpu.sync_copy(sv, s_sh.at[pl.ds(nb + u * UCH, UCH)])
    plsc.subcore_barrier()

    ghalf = g_hbm.at[c]

    @pl.loop(0, KSTEPS)
    def _(t):
        for h in range(2):
            hb = s * EBT + h * (EBT // 2)
            pltpu.sync_copy(row_hbm.at[pl.ds(hb, EBT // 2)], row_v)
            pltpu.sync_copy(col_hbm.at[pl.ds(hb, EBT // 2)], col_v)
            pltpu.async_copy(ghalf.at[row_v.at[0]], bufa, sga)
            pltpu.async_copy(ghalf.at[row_v.at[1]], bufb, sgb)

            @pl.loop(0, 5)
            def _(j):
                b = 4 * j

                @pl.when(j > 0)
                def _():
                    pltpu.make_async_copy(bufc, s_sh.at[col_v.at[0]], ssc).wait()
                    pltpu.make_async_copy(bufd, s_sh.at[col_v.at[0]], ssd).wait()

                pltpu.async_copy(ghalf.at[row_v.at[b + 2]], bufc, sgc)
                pltpu.async_copy(ghalf.at[row_v.at[b + 3]], bufd, sgd)
                pltpu.make_async_copy(ghalf.at[row_v.at[0]], bufa, sga).wait()
                sa = pltpu.async_copy(bufa, s_sh.at[col_v.at[b]], ssa, add=True)
                pltpu.make_async_copy(ghalf.at[row_v.at[0]], bufb, sgb).wait()
                sb = pltpu.async_copy(bufb, s_sh.at[col_v.at[b + 1]], ssb,
                                      add=True)

                @pl.when(j < 4)
                def _():
                    sa.wait()
                    sb.wait()
                    pltpu.async_copy(ghalf.at[row_v.at[b + 4]], bufa, sga)
                    pltpu.async_copy(ghalf.at[row_v.at[b + 5]], bufb, sgb)

                pltpu.make_async_copy(ghalf.at[row_v.at[0]], bufc, sgc).wait()
                pltpu.async_copy(bufc, s_sh.at[col_v.at[b + 2]], ssc, add=True)
                pltpu.make_async_copy(ghalf.at[row_v.at[0]], bufd, sgd).wait()
                pltpu.async_copy(bufd, s_sh.at[col_v.at[b + 3]], ssd, add=True)

            pltpu.make_async_copy(bufa, s_sh.at[col_v.at[0]], ssa).wait()
            pltpu.make_async_copy(bufb, s_sh.at[col_v.at[0]], ssb).wait()
            pltpu.make_async_copy(bufc, s_sh.at[col_v.at[0]], ssc).wait()
            pltpu.make_async_copy(bufd, s_sh.at[col_v.at[0]], ssd).wait()

        plsc.subcore_barrier()

        for u in range(4):
            rb = nb + u * UCH
            pltpu.sync_copy(s_sh.at[pl.ds(rb, UCH)], sv)
            pltpu.sync_copy(g_hbm.at[c, pl.ds(rb, UCH)], gv)
            pltpu.sync_copy(g0_hbm.at[c, pl.ds(rb, UCH)], g0v)
            pltpu.sync_copy(a_hbm.at[pl.ds(rb, UCH)], av)

            @pl.loop(0, UCH)
            def _(r):
                a = av[r]
                for f in range(2):
                    fsl = pl.ds(f * 16, 16)
                    gv[r, fsl] = (a * (sv[r, fsl] + gv[r, fsl])
                                  + ALPHA * g0v[r, fsl])
                    sv[r, fsl] = zero

            pltpu.sync_copy(gv, g_hbm.at[c, pl.ds(rb, UCH)])
            pltpu.sync_copy(sv, s_sh.at[pl.ds(rb, UCH)])

        plsc.subcore_barrier()


def kernel(x, edge_index, W_in, b_in, W_out, b_out):
    npad = EP - NEDGE
    padi = jnp.arange(npad, dtype=jnp.int32)
    row = jnp.concatenate([edge_index[0], padi % NNODE]).reshape(EP // 512, 512)
    col = jnp.concatenate(
        [edge_index[1], NNODE + padi % (NP - NNODE)]).reshape(EP // 512, 512)

    h0 = _linin(x, W_in, b_in.reshape(1, HD))
    g0h, a16, inv16 = _prep_sc(col, h0)
    g4h = _appnp_sc(g0h, a16, row, col)
    out = _fin(g4h, inv16, W_out, b_out.reshape(1, NCLS))
    return out[:NNODE]
